# baseline (speedup 1.0000x reference)
import jax
import jax.numpy as jnp
from jax import lax
from jax.experimental import pallas as pl

K = 8192
D = 32
BETA_C = 0.25
TN = 512
KC = 4096
EPS = 1e-10


def _vq_body(z_ref, e_ref, zq_ref, idx_ref, loss_ref, perp_ref, counts_ref):
    i = pl.program_id(0)
    nsteps = pl.num_programs(0)
    n_total = TN * nsteps

    @pl.when(i == 0)
    def _init():
        counts_ref[...] = jnp.zeros_like(counts_ref)
        loss_ref[...] = jnp.zeros((1, 1), jnp.float32)
        perp_ref[...] = jnp.zeros((1, 1), jnp.float32)

    zb = z_ref[...]
    zn = jnp.sum(zb * zb, axis=1, keepdims=True)
    iota = lax.broadcasted_iota(jnp.int32, (TN, KC), 1)

    bestq = None
    bidx = None
    for c in range(K // KC):
        e_c = e_ref[pl.ds(c * KC, KC), :]
        en = jnp.sum(e_c * e_c, axis=1)[None, :]
        s2 = lax.dot_general(zb, e_c * (-2.0), (((1,), (1,)), ((), ())),
                             preferred_element_type=jnp.float32)
        d = (zn + en) + s2
        m = jnp.min(d, axis=1, keepdims=True)
        li = jnp.min(jnp.where(d <= m, iota, jnp.int32(2 ** 30)),
                     axis=1, keepdims=True) + c * KC
        if c == 0:
            best = m
            bidx = li
        else:
            take = m < bestq
            best = jnp.where(take, m, bestq)
            bidx = jnp.where(take, li, bidx)
        bestq = best.astype(jnp.bfloat16).astype(jnp.float32)

    zq = jnp.zeros((TN, D), jnp.float32)
    for c in range(K // KC):
        e_c = e_ref[pl.ds(c * KC, KC), :]
        oh = (iota == (bidx - c * KC)).astype(jnp.float32)
        zq = zq + lax.dot_general(oh, e_c, (((1,), (0,)), ((), ())),
                                  preferred_element_type=jnp.float32)
        counts_ref[0:1, pl.ds(c * KC, KC)] += jnp.sum(oh, axis=0, keepdims=True)

    zqst = zb + (zq - zb)
    zq_ref[...] = zqst
    idx_ref[...] = bidx
    loss_ref[...] += jnp.sum((zqst - zb) ** 2, keepdims=True)

    @pl.when(i == nsteps - 1)
    def _finalize():
        loss_ref[...] = loss_ref[...] * (BETA_C / (n_total * D))
        p = counts_ref[...] * (1.0 / n_total)
        ent = jnp.sum(p * jnp.log(p + EPS), keepdims=True)
        perp_ref[...] = jnp.exp(-ent)


def kernel(z, embed_weight):
    zf = z.reshape(-1, D)
    n = zf.shape[0]
    nb = n // TN
    zq, idx, loss, perp, _counts = pl.pallas_call(
        _vq_body,
        grid=(nb,),
        in_specs=[
            pl.BlockSpec((TN, D), lambda i: (i, 0)),
            pl.BlockSpec((K, D), lambda i: (0, 0)),
        ],
        out_specs=[
            pl.BlockSpec((TN, D), lambda i: (i, 0)),
            pl.BlockSpec((TN, 1), lambda i: (i, 0)),
            pl.BlockSpec((1, 1), lambda i: (0, 0)),
            pl.BlockSpec((1, 1), lambda i: (0, 0)),
            pl.BlockSpec((1, K), lambda i: (0, 0)),
        ],
        out_shape=[
            jax.ShapeDtypeStruct((n, D), jnp.float32),
            jax.ShapeDtypeStruct((n, 1), jnp.int32),
            jax.ShapeDtypeStruct((1, 1), jnp.float32),
            jax.ShapeDtypeStruct((1, 1), jnp.float32),
            jax.ShapeDtypeStruct((1, K), jnp.float32),
        ],
    )(zf, embed_weight)
    return (zq.reshape(z.shape), loss[0, 0], idx[:, 0], perp[0, 0])


# speedup vs baseline: 1.5075x; 1.5075x over previous
import functools

import jax
import jax.numpy as jnp
from jax import lax
from jax.experimental import pallas as pl
from jax.experimental.pallas import tpu as pltpu
from jax.experimental.pallas import tpu_sc as plsc

K = 8192
D = 32
BETA_C = 0.25
TN = 512
KC = 4096
EPS = 1e-10

NW = 32
CH = 4
CB = 128
BPW = CH * CB


def _argmin_body(z_ref, e_ref, idx_ref, loss_ref):
    i = pl.program_id(0)
    nsteps = pl.num_programs(0)
    n_total = TN * nsteps

    @pl.when(i == 0)
    def _init():
        loss_ref[...] = jnp.zeros((1, 1), jnp.float32)

    zb = z_ref[...]
    zn = jnp.sum(zb * zb, axis=1, keepdims=True)

    bestq = None
    bestx = None
    bidx = None
    for c in range(K // KC):
        e_c = e_ref[pl.ds(c * KC, KC), :]
        en = jnp.sum(e_c * e_c, axis=1)[None, :]
        s = lax.dot_general(zb, e_c, (((1,), (1,)), ((), ())),
                            preferred_element_type=jnp.float32)
        d = (zn + en) - 2.0 * s
        m = jnp.min(d, axis=1, keepdims=True)
        iota = lax.broadcasted_iota(jnp.int32, (TN, KC), 1) + c * KC
        li = jnp.min(jnp.where(d <= m, iota, jnp.int32(2 ** 30)),
                     axis=1, keepdims=True)
        if c == 0:
            bestx = m
            bidx = li
        else:
            take = m < bestq
            bestx = jnp.where(take, m, bestx)
            bidx = jnp.where(take, li, bidx)
        bestq = bestx.astype(jnp.bfloat16).astype(jnp.float32)

    idx_ref[...] = bidx
    loss_ref[...] += jnp.sum(bestx, keepdims=True)

    @pl.when(i == nsteps - 1)
    def _finalize():
        loss_ref[...] = loss_ref[...] * (BETA_C / (n_total * D))


def _sc_gather_hist(ebf, idx3, ones_in, zeros_in):
    mesh = plsc.VectorSubcoreMesh(core_axis_name="c", subcore_axis_name="s")

    @functools.partial(
        pl.kernel,
        mesh=mesh,
        out_type=[
            jax.ShapeDtypeStruct((NW * BPW, 128), jnp.float32),
            jax.ShapeDtypeStruct((2, K, 128), jnp.float32),
        ],
        scratch_types=[
            pltpu.VMEM((CH, CB), jnp.int32),
            pltpu.VMEM((CB, 128), jnp.float32),
            pltpu.VMEM((CB, 128), jnp.float32),
            pltpu.VMEM_SHARED((K, 128), jnp.float32),
            pltpu.SemaphoreType.DMA,
        ],
    )
    def body(ebf_hbm, idx_hbm, ones_hbm, zeros_hbm, zq_hbm, cnt_hbm,
             idx_v, rows_v, ones_v, hist_sh, sem):
        cid = lax.axis_index("c")
        sid = lax.axis_index("s")
        wid = sid * 2 + cid
        base = wid * BPW

        pltpu.sync_copy(idx_hbm.at[wid], idx_v)
        pltpu.sync_copy(ones_hbm, ones_v)
        pltpu.sync_copy(zeros_hbm.at[pl.ds(sid * (K // 16), K // 16)],
                        hist_sh.at[pl.ds(sid * (K // 16), K // 16)])
        plsc.subcore_barrier()

        @pl.loop(0, CH)
        def _(c):
            pltpu.async_copy(ebf_hbm.at[idx_v.at[c]], rows_v, sem).wait()
            pltpu.sync_copy(rows_v, zq_hbm.at[pl.ds(base + c * CB, CB)])
            pltpu.sync_copy(ones_v, hist_sh.at[idx_v.at[c]], add=True)

        plsc.subcore_barrier()
        pltpu.sync_copy(hist_sh.at[pl.ds(sid * (K // 16), K // 16)],
                        cnt_hbm.at[cid, pl.ds(sid * (K // 16), K // 16)])

    return body(ebf, idx3, ones_in, zeros_in)


def _finish_body(cnt_ref, zqp_ref, zq_ref, perp_ref):
    zq_ref[...] = zqp_ref[:, 0:D]
    c0 = cnt_ref[0, :, 0:1]
    c1 = cnt_ref[1, :, 0:1]
    p = (c0 + c1) * (1.0 / (NW * BPW))
    ent = jnp.sum(p * jnp.log(p + EPS), keepdims=True)
    perp_ref[...] = jnp.exp(-ent).reshape(1, 1)


def kernel(z, embed_weight):
    zf = z.reshape(-1, D)
    n = zf.shape[0]
    nb = n // TN
    idx, loss = pl.pallas_call(
        _argmin_body,
        grid=(nb,),
        in_specs=[
            pl.BlockSpec((TN, D), lambda i: (i, 0)),
            pl.BlockSpec((K, D), lambda i: (0, 0)),
        ],
        out_specs=[
            pl.BlockSpec((TN, 1), lambda i: (i, 0)),
            pl.BlockSpec((1, 1), lambda i: (0, 0)),
        ],
        out_shape=[
            jax.ShapeDtypeStruct((n, 1), jnp.int32),
            jax.ShapeDtypeStruct((1, 1), jnp.float32),
        ],
    )(zf, embed_weight)

    ebf = embed_weight.astype(jnp.bfloat16).astype(jnp.float32)
    ebf_pad = jnp.pad(ebf, ((0, 0), (0, 128 - D)))
    idx3 = idx[:, 0].reshape(NW, CH, CB)
    ones_in = jnp.ones((CB, 128), jnp.float32)
    zeros_in = jnp.zeros((K, 128), jnp.float32)
    zq_pad, counts = _sc_gather_hist(ebf_pad, idx3, ones_in, zeros_in)

    zq, perp = pl.pallas_call(
        _finish_body,
        in_specs=[pl.BlockSpec((2, K, 128), lambda: (0, 0, 0)),
                  pl.BlockSpec((n, 128), lambda: (0, 0))],
        out_specs=[pl.BlockSpec((n, D), lambda: (0, 0)),
                   pl.BlockSpec((1, 1), lambda: (0, 0))],
        out_shape=[jax.ShapeDtypeStruct((n, D), jnp.float32),
                   jax.ShapeDtypeStruct((1, 1), jnp.float32)],
    )(counts, zq_pad)

    return (zq.reshape(z.shape), loss[0, 0], idx[:, 0], perp[0, 0])


# speedup vs baseline: 1.6173x; 1.0729x over previous
import functools

import jax
import jax.numpy as jnp
from jax import lax
from jax.experimental import pallas as pl
from jax.experimental.pallas import tpu as pltpu
from jax.experimental.pallas import tpu_sc as plsc

K = 8192
D = 32
BETA_C = 0.25
TN = 1024
KC = 4096
EPS = 1e-10

NW = 32
CH = 4
CB = 128
BPW = CH * CB


def _argmin_body(z_ref, e_ref, idx_ref, loss_ref):
    i = pl.program_id(0)
    nsteps = pl.num_programs(0)
    n_total = TN * nsteps

    @pl.when(i == 0)
    def _init():
        loss_ref[...] = jnp.zeros((1, 1), jnp.float32)

    zb = z_ref[...]
    zn = jnp.sum(zb * zb, axis=1, keepdims=True)

    bestq = None
    bestx = None
    bidx = None
    for c in range(K // KC):
        e_c = e_ref[pl.ds(c * KC, KC), :]
        en = jnp.sum(e_c * e_c, axis=1)[None, :]
        s = lax.dot_general(zb, e_c, (((1,), (1,)), ((), ())),
                            preferred_element_type=jnp.float32)
        d = (zn + en) - 2.0 * s
        m = jnp.min(d, axis=1, keepdims=True)
        iota = lax.broadcasted_iota(jnp.int32, (TN, KC), 1) + c * KC
        li = jnp.min(jnp.where(d <= m, iota, jnp.int32(2 ** 30)),
                     axis=1, keepdims=True)
        if c == 0:
            bestx = m
            bidx = li
        else:
            take = m < bestq
            bestx = jnp.where(take, m, bestx)
            bidx = jnp.where(take, li, bidx)
        bestq = bestx.astype(jnp.bfloat16).astype(jnp.float32)

    idx_ref[...] = bidx
    loss_ref[...] += jnp.sum(bestx, keepdims=True)

    @pl.when(i == nsteps - 1)
    def _finalize():
        loss_ref[...] = loss_ref[...] * (BETA_C / (n_total * D))


def _sc_gather_hist(ebf, idx3, ones_in, zeros_in):
    mesh = plsc.VectorSubcoreMesh(core_axis_name="c", subcore_axis_name="s")

    @functools.partial(
        pl.kernel,
        mesh=mesh,
        out_type=[
            jax.ShapeDtypeStruct((NW * BPW, 128), jnp.float32),
            jax.ShapeDtypeStruct((2, K, 128), jnp.float32),
        ],
        scratch_types=[
            pltpu.VMEM((CH, CB), jnp.int32),
            pltpu.VMEM((CB, 128), jnp.float32),
            pltpu.VMEM((CB, 128), jnp.float32),
            pltpu.VMEM_SHARED((K, 128), jnp.float32),
            pltpu.SemaphoreType.DMA,
        ],
    )
    def body(ebf_hbm, idx_hbm, ones_hbm, zeros_hbm, zq_hbm, cnt_hbm,
             idx_v, rows_v, ones_v, hist_sh, sem):
        cid = lax.axis_index("c")
        sid = lax.axis_index("s")
        wid = sid * 2 + cid
        base = wid * BPW

        pltpu.sync_copy(idx_hbm.at[wid], idx_v)
        pltpu.sync_copy(ones_hbm, ones_v)
        pltpu.sync_copy(zeros_hbm.at[pl.ds(sid * (K // 16), K // 16)],
                        hist_sh.at[pl.ds(sid * (K // 16), K // 16)])
        plsc.subcore_barrier()

        @pl.loop(0, CH)
        def _(c):
            pltpu.async_copy(ebf_hbm.at[idx_v.at[c]], rows_v, sem).wait()
            pltpu.sync_copy(rows_v, zq_hbm.at[pl.ds(base + c * CB, CB)])
            pltpu.sync_copy(ones_v, hist_sh.at[idx_v.at[c]], add=True)

        plsc.subcore_barrier()
        pltpu.sync_copy(hist_sh.at[pl.ds(sid * (K // 16), K // 16)],
                        cnt_hbm.at[cid, pl.ds(sid * (K // 16), K // 16)])

    return body(ebf, idx3, ones_in, zeros_in)


def _finish_body(cnt_ref, zqp_ref, zq_ref, perp_ref):
    zq_ref[...] = zqp_ref[:, 0:D]
    c0 = cnt_ref[0, :, 0:1]
    c1 = cnt_ref[1, :, 0:1]
    p = (c0 + c1) * (1.0 / (NW * BPW))
    ent = jnp.sum(p * jnp.log(p + EPS), keepdims=True)
    perp_ref[...] = jnp.exp(-ent).reshape(1, 1)


def kernel(z, embed_weight):
    zf = z.reshape(-1, D)
    n = zf.shape[0]
    nb = n // TN
    idx, loss = pl.pallas_call(
        _argmin_body,
        grid=(nb,),
        in_specs=[
            pl.BlockSpec((TN, D), lambda i: (i, 0)),
            pl.BlockSpec((K, D), lambda i: (0, 0)),
        ],
        out_specs=[
            pl.BlockSpec((TN, 1), lambda i: (i, 0)),
            pl.BlockSpec((1, 1), lambda i: (0, 0)),
        ],
        out_shape=[
            jax.ShapeDtypeStruct((n, 1), jnp.int32),
            jax.ShapeDtypeStruct((1, 1), jnp.float32),
        ],
    )(zf, embed_weight)

    ebf = embed_weight.astype(jnp.bfloat16).astype(jnp.float32)
    ebf_pad = jnp.pad(ebf, ((0, 0), (0, 128 - D)))
    idx3 = idx[:, 0].reshape(NW, CH, CB)
    ones_in = jnp.ones((CB, 128), jnp.float32)
    zeros_in = jnp.zeros((K, 128), jnp.float32)
    zq_pad, counts = _sc_gather_hist(ebf_pad, idx3, ones_in, zeros_in)

    zq, perp = pl.pallas_call(
        _finish_body,
        in_specs=[pl.BlockSpec((2, K, 128), lambda: (0, 0, 0)),
                  pl.BlockSpec((n, 128), lambda: (0, 0))],
        out_specs=[pl.BlockSpec((n, D), lambda: (0, 0)),
                   pl.BlockSpec((1, 1), lambda: (0, 0))],
        out_shape=[jax.ShapeDtypeStruct((n, D), jnp.float32),
                   jax.ShapeDtypeStruct((1, 1), jnp.float32)],
    )(counts, zq_pad)

    return (zq.reshape(z.shape), loss[0, 0], idx[:, 0], perp[0, 0])


# speedup vs baseline: 1.6608x; 1.0269x over previous
import functools

import jax
import jax.numpy as jnp
from jax import lax
from jax.experimental import pallas as pl
from jax.experimental.pallas import tpu as pltpu
from jax.experimental.pallas import tpu_sc as plsc

K = 8192
D = 32
BETA_C = 0.25
TN = 2048
KC = 4096
EPS = 1e-10

NW = 32
CH = 4
CB = 128
BPW = CH * CB


def _argmin_body(z_ref, e_ref, idx_ref, loss_ref):
    i = pl.program_id(0)
    nsteps = pl.num_programs(0)
    n_total = TN * nsteps

    @pl.when(i == 0)
    def _init():
        loss_ref[...] = jnp.zeros((1, 1), jnp.float32)

    zb = z_ref[...]
    zn = jnp.sum(zb * zb, axis=1, keepdims=True)

    bestq = None
    bestx = None
    bidx = None
    for c in range(K // KC):
        e_c = e_ref[pl.ds(c * KC, KC), :]
        en = jnp.sum(e_c * e_c, axis=1)[None, :]
        s = lax.dot_general(zb, e_c, (((1,), (1,)), ((), ())),
                            preferred_element_type=jnp.float32)
        d = (zn + en) - 2.0 * s
        m = jnp.min(d, axis=1, keepdims=True)
        iota = lax.broadcasted_iota(jnp.int32, (TN, KC), 1) + c * KC
        li = jnp.min(jnp.where(d <= m, iota, jnp.int32(2 ** 30)),
                     axis=1, keepdims=True)
        if c == 0:
            bestx = m
            bidx = li
        else:
            take = m < bestq
            bestx = jnp.where(take, m, bestx)
            bidx = jnp.where(take, li, bidx)
        bestq = bestx.astype(jnp.bfloat16).astype(jnp.float32)

    idx_ref[...] = bidx
    loss_ref[...] += jnp.sum(bestx, keepdims=True)

    @pl.when(i == nsteps - 1)
    def _finalize():
        loss_ref[...] = loss_ref[...] * (BETA_C / (n_total * D))


def _sc_gather_hist(ebf, idx3, ones_in, zeros_in):
    mesh = plsc.VectorSubcoreMesh(core_axis_name="c", subcore_axis_name="s")

    @functools.partial(
        pl.kernel,
        mesh=mesh,
        out_type=[
            jax.ShapeDtypeStruct((NW * BPW, 128), jnp.float32),
            jax.ShapeDtypeStruct((2, K, 128), jnp.float32),
        ],
        scratch_types=[
            pltpu.VMEM((CH, CB), jnp.int32),
            pltpu.VMEM((CB, 128), jnp.float32),
            pltpu.VMEM((CB, 128), jnp.float32),
            pltpu.VMEM_SHARED((K, 128), jnp.float32),
            pltpu.SemaphoreType.DMA,
        ],
    )
    def body(ebf_hbm, idx_hbm, ones_hbm, zeros_hbm, zq_hbm, cnt_hbm,
             idx_v, rows_v, ones_v, hist_sh, sem):
        cid = lax.axis_index("c")
        sid = lax.axis_index("s")
        wid = sid * 2 + cid
        base = wid * BPW

        pltpu.sync_copy(idx_hbm.at[wid], idx_v)
        pltpu.sync_copy(ones_hbm, ones_v)
        pltpu.sync_copy(zeros_hbm.at[pl.ds(sid * (K // 16), K // 16)],
                        hist_sh.at[pl.ds(sid * (K // 16), K // 16)])
        plsc.subcore_barrier()

        @pl.loop(0, CH)
        def _(c):
            pltpu.async_copy(ebf_hbm.at[idx_v.at[c]], rows_v, sem).wait()
            pltpu.sync_copy(rows_v, zq_hbm.at[pl.ds(base + c * CB, CB)])
            pltpu.sync_copy(ones_v, hist_sh.at[idx_v.at[c]], add=True)

        plsc.subcore_barrier()
        pltpu.sync_copy(hist_sh.at[pl.ds(sid * (K // 16), K // 16)],
                        cnt_hbm.at[cid, pl.ds(sid * (K // 16), K // 16)])

    return body(ebf, idx3, ones_in, zeros_in)


def _finish_body(cnt_ref, zqp_ref, zq_ref, perp_ref):
    zq_ref[...] = zqp_ref[:, 0:D]
    c0 = cnt_ref[0, :, 0:1]
    c1 = cnt_ref[1, :, 0:1]
    p = (c0 + c1) * (1.0 / (NW * BPW))
    ent = jnp.sum(p * jnp.log(p + EPS), keepdims=True)
    perp_ref[...] = jnp.exp(-ent).reshape(1, 1)


def kernel(z, embed_weight):
    zf = z.reshape(-1, D)
    n = zf.shape[0]
    nb = n // TN
    idx, loss = pl.pallas_call(
        _argmin_body,
        grid=(nb,),
        in_specs=[
            pl.BlockSpec((TN, D), lambda i: (i, 0)),
            pl.BlockSpec((K, D), lambda i: (0, 0)),
        ],
        out_specs=[
            pl.BlockSpec((TN, 1), lambda i: (i, 0)),
            pl.BlockSpec((1, 1), lambda i: (0, 0)),
        ],
        out_shape=[
            jax.ShapeDtypeStruct((n, 1), jnp.int32),
            jax.ShapeDtypeStruct((1, 1), jnp.float32),
        ],
    )(zf, embed_weight)

    ebf = embed_weight.astype(jnp.bfloat16).astype(jnp.float32)
    ebf_pad = jnp.pad(ebf, ((0, 0), (0, 128 - D)))
    idx3 = idx[:, 0].reshape(NW, CH, CB)
    ones_in = jnp.ones((CB, 128), jnp.float32)
    zeros_in = jnp.zeros((K, 128), jnp.float32)
    zq_pad, counts = _sc_gather_hist(ebf_pad, idx3, ones_in, zeros_in)

    zq, perp = pl.pallas_call(
        _finish_body,
        in_specs=[pl.BlockSpec((2, K, 128), lambda: (0, 0, 0)),
                  pl.BlockSpec((n, 128), lambda: (0, 0))],
        out_specs=[pl.BlockSpec((n, D), lambda: (0, 0)),
                   pl.BlockSpec((1, 1), lambda: (0, 0))],
        out_shape=[jax.ShapeDtypeStruct((n, D), jnp.float32),
                   jax.ShapeDtypeStruct((1, 1), jnp.float32)],
    )(counts, zq_pad)

    return (zq.reshape(z.shape), loss[0, 0], idx[:, 0], perp[0, 0])
